# baseline (speedup 1.0000x reference)
import functools

import jax
import jax.numpy as jnp
from jax import lax
from jax.experimental import pallas as pl
from jax.experimental.pallas import tpu as pltpu
from jax.experimental.pallas import tpu_sc as plsc

NC = 2
NS = 16
NW = NC * NS
K = 128
RB = 1000


def _sc_mesh():
    return plsc.VectorSubcoreMesh(core_axis_name="c", subcore_axis_name="s")


@functools.partial(jax.jit, static_argnums=(2,))
def _degree_partials(c2, w2, n):
    ept = c2.shape[1]

    @functools.partial(
        pl.kernel,
        out_type=jax.ShapeDtypeStruct((NW, n), jnp.float32),
        mesh=_sc_mesh(),
        scratch_types=[
            pltpu.VMEM((ept,), jnp.int32),
            pltpu.VMEM((ept,), jnp.float32),
            pltpu.VMEM((n,), jnp.float32),
        ],
        compiler_params=pltpu.CompilerParams(
            needs_layout_passes=False, use_tc_tiling_on_sc=False),
    )
    def degk(c_hbm, w_hbm, out_hbm, cidx, wbuf, dloc):
        cid = lax.axis_index("c")
        sid = lax.axis_index("s")
        wid = cid * NS + sid
        pltpu.sync_copy(c_hbm.at[wid], cidx)
        pltpu.sync_copy(w_hbm.at[wid], wbuf)
        zv = jnp.zeros((16,), jnp.float32)

        def zbody(i, _):
            dloc[pl.ds(i * 16, 16)] = zv
            return 0

        lax.fori_loop(0, n // 16, zbody, 0)

        def abody(i, _):
            cv = cidx[pl.ds(i * 16, 16)]
            wv = wbuf[pl.ds(i * 16, 16)]
            plsc.addupdate_scatter(dloc, [cv], wv)
            return 0

        lax.fori_loop(0, ept // 16, abody, 0)
        pltpu.sync_copy(dloc, out_hbm.at[wid])

    return degk(c2, w2)


@functools.partial(jax.jit, static_argnums=(4, 5))
def _message_partials(xwp, r2, c3, w2, n, h):
    ept = r2.shape[1]
    nchunk = ept // K
    stripe = n // NS
    assert nchunk % 3 == 2, nchunk
    nt = (nchunk - 2) // 3

    @functools.partial(
        pl.kernel,
        out_type=jax.ShapeDtypeStruct((NC, n, h), jnp.float32),
        mesh=_sc_mesh(),
        scratch_types=[
            pltpu.VMEM((nchunk, K), jnp.int32),
            pltpu.VMEM((ept,), jnp.int32),
            pltpu.VMEM((ept,), jnp.float32),
            pltpu.VMEM((K, h), jnp.float32),
            pltpu.VMEM((K, h), jnp.float32),
            pltpu.VMEM((K, h), jnp.float32),
            pltpu.VMEM((stripe // 25, h), jnp.float32),
            pltpu.VMEM_SHARED((n, h), jnp.float32),
            pltpu.SemaphoreType.DMA,
            pltpu.SemaphoreType.DMA,
            pltpu.SemaphoreType.DMA,
            pltpu.SemaphoreType.DMA,
            pltpu.SemaphoreType.DMA,
            pltpu.SemaphoreType.DMA,
        ],
        compiler_params=pltpu.CompilerParams(
            needs_layout_passes=False, use_tc_tiling_on_sc=False),
    )
    def msgk(xwp_hbm, r_hbm, c_hbm, w_hbm, out_hbm,
             cidx, ridx, wbuf, rows0, rows1, rows2, zbuf, acc_sh,
             g0, g1, g2, s0, s1, s2):
        rows = (rows0, rows1, rows2)
        gsem = (g0, g1, g2)
        ssem = (s0, s1, s2)
        cid = lax.axis_index("c")
        sid = lax.axis_index("s")
        wid = cid * NS + sid
        pltpu.sync_copy(r_hbm.at[wid], ridx)
        pltpu.sync_copy(c_hbm.at[wid], cidx)
        pltpu.sync_copy(w_hbm.at[wid], wbuf)

        zrows = stripe // 25
        zv = jnp.zeros((16,), jnp.float32)

        def zbody(i, _):
            for jj in range(h // 16):
                zbuf[i, pl.ds(jj * 16, 16)] = zv
            return 0

        lax.fori_loop(0, zrows, zbody, 0)

        def zcopy(i, _):
            pltpu.sync_copy(
                zbuf, acc_sh.at[pl.ds(sid * stripe + i * zrows, zrows)])
            return 0

        lax.fori_loop(0, 25, zcopy, 0)
        plsc.subcore_barrier()

        def start_gather(m, b):
            pltpu.async_copy(
                xwp_hbm.at[ridx.at[pl.ds(m * K, K)]], rows[b], gsem[b])

        def wait_gather(m, b):
            pltpu.make_async_copy(
                xwp_hbm.at[ridx.at[pl.ds(m * K, K)]], rows[b], gsem[b]).wait()

        def start_scatter(m, b):
            pltpu.async_copy(rows[b], acc_sh.at[cidx.at[m]], ssem[b],
                             add=True)

        def wait_scatter(m, b):
            pltpu.make_async_copy(
                rows[b], acc_sh.at[cidx.at[m]], ssem[b]).wait()

        def scale(b, m):
            rb = rows[b]

            def edge_body(e):
                wspl = plsc.load_gather(
                    wbuf, [jnp.full((16,), m * K + e, jnp.int32)])
                for jj in range(h // 16):
                    rb[e, pl.ds(jj * 16, 16)] = (
                        rb[e, pl.ds(jj * 16, 16)] * wspl)

            plsc.parallel_loop(0, K, 1, unroll=8)(edge_body)

        start_gather(0, 0)
        start_gather(1, 1)
        wait_gather(0, 0)
        scale(0, 0)
        start_gather(2, 2)
        start_scatter(0, 0)

        def triple(t, _):
            for p in range(3):
                m = 3 * t + 1 + p
                b = (1 + p) % 3
                bn = (b + 2) % 3
                wait_gather(m, b)
                scale(b, m)
                wait_scatter(m - 1, bn)
                if p == 2:
                    @pl.when(t < nt - 1)
                    def _():
                        start_gather(m + 2, bn)
                else:
                    start_gather(m + 2, bn)
                start_scatter(m, b)
            return 0

        lax.fori_loop(0, nt, triple, 0)
        m_last = nchunk - 1
        b_last = m_last % 3
        wait_gather(m_last, b_last)
        scale(b_last, m_last)
        wait_scatter(m_last - 1, (b_last + 2) % 3)
        start_scatter(m_last, b_last)
        wait_scatter(m_last, b_last)
        plsc.subcore_barrier()
        pltpu.sync_copy(acc_sh.at[pl.ds(sid * stripe, stripe)],
                        out_hbm.at[cid, pl.ds(sid * stripe, stripe)])

    return msgk(xwp, r2, c3, w2)


def _tc_stage1(degp_t, x, w1):
    n, d = x.shape
    hh = w1.shape[1]
    nw = degp_t.shape[1]
    grid = (n // RB,)

    def body(degp_ref, x_ref, w_ref, xwp_ref, dinv_ref):
        deg = jnp.sum(degp_ref[...], axis=1, keepdims=True) + 1.0
        dinv = jnp.where(deg == 0.0, 0.0, lax.rsqrt(jnp.maximum(deg, 1e-30)))
        xw = jnp.dot(x_ref[...], w_ref[...],
                     preferred_element_type=jnp.float32)
        xwp_ref[...] = xw * dinv
        dinv_ref[...] = dinv

    return pl.pallas_call(
        body,
        grid=grid,
        in_specs=[
            pl.BlockSpec((RB, nw), lambda i: (i, 0)),
            pl.BlockSpec((RB, d), lambda i: (i, 0)),
            pl.BlockSpec((d, hh), lambda i: (0, 0)),
        ],
        out_specs=[
            pl.BlockSpec((RB, hh), lambda i: (i, 0)),
            pl.BlockSpec((RB, 1), lambda i: (i, 0)),
        ],
        out_shape=[
            jax.ShapeDtypeStruct((n, hh), jnp.float32),
            jax.ShapeDtypeStruct((n, 1), jnp.float32),
        ],
    )(degp_t, x, w1)


def _tc_stage2(acc, xwp, dinv, b1, w2):
    n, hh = xwp.shape
    grid = (n // RB,)

    def body(acc_ref, xwp_ref, dinv_ref, b_ref, w_ref, out_ref):
        pre = acc_ref[0] + acc_ref[1] + xwp_ref[...]
        hv = pre * dinv_ref[...] + b_ref[...]
        hv = jnp.where(hv >= 0.0, hv, 0.01 * hv)
        xw = jnp.dot(hv, w_ref[...], preferred_element_type=jnp.float32)
        out_ref[...] = xw * dinv_ref[...]

    return pl.pallas_call(
        body,
        grid=grid,
        in_specs=[
            pl.BlockSpec((NC, RB, hh), lambda i: (0, i, 0)),
            pl.BlockSpec((RB, hh), lambda i: (i, 0)),
            pl.BlockSpec((RB, 1), lambda i: (i, 0)),
            pl.BlockSpec((1, hh), lambda i: (0, 0)),
            pl.BlockSpec((hh, hh), lambda i: (0, 0)),
        ],
        out_specs=pl.BlockSpec((RB, hh), lambda i: (i, 0)),
        out_shape=jax.ShapeDtypeStruct((n, hh), jnp.float32),
    )(acc, xwp, dinv, b1, w2)


def _tc_stage3(acc, xwp, dinv, b2, wm1, bm1, wm2, bm2):
    n, hh = xwp.shape
    c = wm2.shape[1]
    grid = (n // RB,)

    def body(acc_ref, xwp_ref, dinv_ref, b2_ref, wm1_ref, bm1_ref,
             wm2_ref, bm2_ref, out_ref):
        h2 = ((acc_ref[0] + acc_ref[1] + xwp_ref[...]) * dinv_ref[...]
              + b2_ref[...])
        t = jnp.dot(h2, wm1_ref[...],
                    preferred_element_type=jnp.float32) + bm1_ref[...]
        t = jnp.where(t > 0.0, t, jnp.exp(jnp.minimum(t, 0.0)) - 1.0)
        lg = jnp.dot(t, wm2_ref[...],
                     preferred_element_type=jnp.float32) + bm2_ref[...]
        m = jnp.max(lg, axis=-1, keepdims=True)
        ev = jnp.exp(lg - m)
        out_ref[...] = ev / jnp.sum(ev, axis=-1, keepdims=True)

    return pl.pallas_call(
        body,
        grid=grid,
        in_specs=[
            pl.BlockSpec((NC, RB, hh), lambda i: (0, i, 0)),
            pl.BlockSpec((RB, hh), lambda i: (i, 0)),
            pl.BlockSpec((RB, 1), lambda i: (i, 0)),
            pl.BlockSpec((1, hh), lambda i: (0, 0)),
            pl.BlockSpec((hh, hh), lambda i: (0, 0)),
            pl.BlockSpec((1, hh), lambda i: (0, 0)),
            pl.BlockSpec((hh, c), lambda i: (0, 0)),
            pl.BlockSpec((1, c), lambda i: (0, 0)),
        ],
        out_specs=pl.BlockSpec((RB, c), lambda i: (i, 0)),
        out_shape=jax.ShapeDtypeStruct((n, c), jnp.float32),
    )(acc, xwp, dinv, b2, wm1, bm1, wm2, bm2)


def kernel(X, edge_index, edge_weight, W1, b1, W2, b2, Wm1, bm1, Wm2, bm2):
    n, d = X.shape
    hh = W1.shape[1]
    c = Wm2.shape[1]
    e = edge_weight.shape[0]
    ept = e // NW
    nchunk = -(-ept // K)
    while nchunk % 3 != 2:
        nchunk += 1
    eptp = nchunk * K
    pad = eptp - ept

    r2 = edge_index[0].reshape(NW, ept)
    c2 = edge_index[1].reshape(NW, ept)
    w2e = edge_weight.reshape(NW, ept)
    rp = jnp.pad(r2, ((0, 0), (0, pad)))
    cp = jnp.pad(c2, ((0, 0), (0, pad)))
    wp = jnp.pad(w2e, ((0, 0), (0, pad)))
    c3 = cp.reshape(NW, eptp // K, K)

    degp = _degree_partials(c2, w2e, n)
    xw1p, dinv = _tc_stage1(degp.T, X, W1)
    acc1 = _message_partials(xw1p, rp, c3, wp, n, hh)
    xw2p = _tc_stage2(acc1, xw1p, dinv, b1.reshape(1, hh), W2)
    acc2 = _message_partials(xw2p, rp, c3, wp, n, hh)
    return _tc_stage3(acc2, xw2p, dinv, b2.reshape(1, hh),
                      Wm1, bm1.reshape(1, hh), Wm2, bm2.reshape(1, c))


# speedup vs baseline: 2.0873x; 2.0873x over previous
import functools

import jax
import jax.numpy as jnp
from jax import lax
from jax.experimental import pallas as pl
from jax.experimental.pallas import tpu as pltpu
from jax.experimental.pallas import tpu_sc as plsc

NC = 2
NS = 16
NW = NC * NS
K = 80
RB = 2000


def _sc_mesh():
    return plsc.VectorSubcoreMesh(core_axis_name="c", subcore_axis_name="s")


@functools.partial(jax.jit, static_argnums=(2,))
def _degree_partials(ei, ew, n):
    ept = ew.shape[0] // NW

    @functools.partial(
        pl.kernel,
        out_type=jax.ShapeDtypeStruct((NW, n), jnp.float32),
        mesh=_sc_mesh(),
        scratch_types=[
            pltpu.VMEM((ept,), jnp.int32),
            pltpu.VMEM((ept,), jnp.float32),
            pltpu.VMEM((n,), jnp.float32),
        ],
        compiler_params=pltpu.CompilerParams(
            needs_layout_passes=False, use_tc_tiling_on_sc=False),
    )
    def degk(ei_hbm, ew_hbm, out_hbm, cidx, wbuf, dloc):
        cid = lax.axis_index("c")
        sid = lax.axis_index("s")
        wid = cid * NS + sid
        pltpu.sync_copy(ei_hbm.at[1, pl.ds(wid * ept, ept)], cidx)
        pltpu.sync_copy(ew_hbm.at[pl.ds(wid * ept, ept)], wbuf)
        zv = jnp.zeros((16,), jnp.float32)

        def zbody(i, _):
            for u in range(4):
                dloc[pl.ds(i * 64 + u * 16, 16)] = zv
            return 0

        lax.fori_loop(0, n // 64, zbody, 0)

        def abody(i, _):
            cv = cidx[pl.ds(i * 16, 16)]
            wv = wbuf[pl.ds(i * 16, 16)]
            plsc.addupdate_scatter(dloc, [cv], wv)
            return 0

        lax.fori_loop(0, ept // 16, abody, 0)
        pltpu.sync_copy(dloc, out_hbm.at[wid])

    return degk(ei, ew)


@functools.partial(jax.jit, static_argnums=(4, 5))
def _message_partials(xwp, ei, c3, ew, n, h):
    ept = ew.shape[0] // NW
    nchunk = ept // K
    stripe = n // NS
    assert nchunk % 3 == 2, nchunk
    nt = (nchunk - 2) // 3

    @functools.partial(
        pl.kernel,
        out_type=jax.ShapeDtypeStruct((NC, n, h), jnp.float32),
        mesh=_sc_mesh(),
        scratch_types=[
            pltpu.VMEM((nchunk, K), jnp.int32),
            pltpu.VMEM((ept,), jnp.int32),
            pltpu.VMEM((ept,), jnp.float32),
            pltpu.VMEM((K, h), jnp.float32),
            pltpu.VMEM((K, h), jnp.float32),
            pltpu.VMEM((K, h), jnp.float32),
            pltpu.VMEM((stripe // 25, h), jnp.float32),
            pltpu.VMEM_SHARED((n, h), jnp.float32),
            pltpu.SemaphoreType.DMA,
            pltpu.SemaphoreType.DMA,
            pltpu.SemaphoreType.DMA,
            pltpu.SemaphoreType.DMA,
            pltpu.SemaphoreType.DMA,
            pltpu.SemaphoreType.DMA,
        ],
        compiler_params=pltpu.CompilerParams(
            needs_layout_passes=False, use_tc_tiling_on_sc=False),
    )
    def msgk(xwp_hbm, ei_hbm, c_hbm, ew_hbm, out_hbm,
             cidx, ridx, wbuf, rows0, rows1, rows2, zbuf, acc_sh,
             g0, g1, g2, s0, s1, s2):
        rows = (rows0, rows1, rows2)
        gsem = (g0, g1, g2)
        ssem = (s0, s1, s2)
        cid = lax.axis_index("c")
        sid = lax.axis_index("s")
        wid = cid * NS + sid
        pltpu.async_copy(ei_hbm.at[0, pl.ds(wid * ept, ept)], ridx, g0)
        pltpu.async_copy(c_hbm.at[wid], cidx, g1)
        pltpu.async_copy(ew_hbm.at[pl.ds(wid * ept, ept)], wbuf, g2)

        zrows = stripe // 25
        zv = jnp.zeros((16,), jnp.float32)

        def zbody(i, _):
            for jj in range(h // 16):
                zbuf[i, pl.ds(jj * 16, 16)] = zv
            return 0

        lax.fori_loop(0, zrows, zbody, 0)

        def zcopy(i, _):
            pltpu.sync_copy(
                zbuf, acc_sh.at[pl.ds(sid * stripe + i * zrows, zrows)])
            return 0

        lax.fori_loop(0, 25, zcopy, 0)
        pltpu.make_async_copy(
            ei_hbm.at[0, pl.ds(wid * ept, ept)], ridx, g0).wait()
        pltpu.make_async_copy(c_hbm.at[wid], cidx, g1).wait()
        pltpu.make_async_copy(
            ew_hbm.at[pl.ds(wid * ept, ept)], wbuf, g2).wait()
        plsc.subcore_barrier()

        def start_gather(m, b):
            pltpu.async_copy(
                xwp_hbm.at[ridx.at[pl.ds(m * K, K)]], rows[b], gsem[b])

        def wait_gather(m, b):
            pltpu.make_async_copy(
                xwp_hbm.at[ridx.at[pl.ds(m * K, K)]], rows[b], gsem[b]).wait()

        def start_scatter(m, b):
            pltpu.async_copy(rows[b], acc_sh.at[cidx.at[m]], ssem[b],
                             add=True)

        def wait_scatter(m, b):
            pltpu.make_async_copy(
                rows[b], acc_sh.at[cidx.at[m]], ssem[b]).wait()

        def scale(b, m):
            rb = rows[b]

            def edge_body(e):
                wspl = plsc.load_gather(
                    wbuf, [jnp.full((16,), m * K + e, jnp.int32)])
                for jj in range(h // 16):
                    rb[e, pl.ds(jj * 16, 16)] = (
                        rb[e, pl.ds(jj * 16, 16)] * wspl)

            plsc.parallel_loop(0, K, 1, unroll=16)(edge_body)

        start_gather(0, 0)
        start_gather(1, 1)
        wait_gather(0, 0)
        scale(0, 0)
        start_gather(2, 2)
        start_scatter(0, 0)

        def triple(t, _):
            for p in range(3):
                m = 3 * t + 1 + p
                b = (1 + p) % 3
                bn = (b + 2) % 3
                wait_gather(m, b)
                scale(b, m)
                wait_scatter(m - 1, bn)
                if p == 2:
                    @pl.when(t < nt - 1)
                    def _():
                        start_gather(m + 2, bn)
                else:
                    start_gather(m + 2, bn)
                start_scatter(m, b)
            return 0

        lax.fori_loop(0, nt, triple, 0)
        m_last = nchunk - 1
        b_last = m_last % 3
        wait_gather(m_last, b_last)
        scale(b_last, m_last)
        wait_scatter(m_last - 1, (b_last + 2) % 3)
        start_scatter(m_last, b_last)
        wait_scatter(m_last, b_last)
        plsc.subcore_barrier()
        pltpu.sync_copy(acc_sh.at[pl.ds(sid * stripe, stripe)],
                        out_hbm.at[cid, pl.ds(sid * stripe, stripe)])

    return msgk(xwp, ei, c3, ew)


def _tc_stage1(degp_t, x, w1):
    n, d = x.shape
    hh = w1.shape[1]
    nw = degp_t.shape[1]
    grid = (n // RB,)

    def body(degp_ref, x_ref, w_ref, xwp_ref, dinv_ref):
        deg = jnp.sum(degp_ref[...], axis=1, keepdims=True) + 1.0
        dinv = jnp.where(deg == 0.0, 0.0, lax.rsqrt(jnp.maximum(deg, 1e-30)))
        xw = jnp.dot(x_ref[...], w_ref[...],
                     preferred_element_type=jnp.float32)
        xwp_ref[...] = xw * dinv
        dinv_ref[...] = dinv

    return pl.pallas_call(
        body,
        grid=grid,
        in_specs=[
            pl.BlockSpec((RB, nw), lambda i: (i, 0)),
            pl.BlockSpec((RB, d), lambda i: (i, 0)),
            pl.BlockSpec((d, hh), lambda i: (0, 0)),
        ],
        out_specs=[
            pl.BlockSpec((RB, hh), lambda i: (i, 0)),
            pl.BlockSpec((RB, 1), lambda i: (i, 0)),
        ],
        out_shape=[
            jax.ShapeDtypeStruct((n, hh), jnp.float32),
            jax.ShapeDtypeStruct((n, 1), jnp.float32),
        ],
    )(degp_t, x, w1)


def _tc_stage2(acc, xwp, dinv, b1, w2):
    n, hh = xwp.shape
    grid = (n // RB,)

    def body(acc_ref, xwp_ref, dinv_ref, b_ref, w_ref, out_ref):
        pre = acc_ref[0] + acc_ref[1] + xwp_ref[...]
        hv = pre * dinv_ref[...] + b_ref[...]
        hv = jnp.where(hv >= 0.0, hv, 0.01 * hv)
        xw = jnp.dot(hv, w_ref[...], preferred_element_type=jnp.float32)
        out_ref[...] = xw * dinv_ref[...]

    return pl.pallas_call(
        body,
        grid=grid,
        in_specs=[
            pl.BlockSpec((NC, RB, hh), lambda i: (0, i, 0)),
            pl.BlockSpec((RB, hh), lambda i: (i, 0)),
            pl.BlockSpec((RB, 1), lambda i: (i, 0)),
            pl.BlockSpec((1, hh), lambda i: (0, 0)),
            pl.BlockSpec((hh, hh), lambda i: (0, 0)),
        ],
        out_specs=pl.BlockSpec((RB, hh), lambda i: (i, 0)),
        out_shape=jax.ShapeDtypeStruct((n, hh), jnp.float32),
    )(acc, xwp, dinv, b1, w2)


def _tc_stage3(acc, xwp, dinv, b2, wm1, bm1, wm2, bm2):
    n, hh = xwp.shape
    c = wm2.shape[1]
    grid = (n // RB,)

    def body(acc_ref, xwp_ref, dinv_ref, b2_ref, wm1_ref, bm1_ref,
             wm2_ref, bm2_ref, out_ref):
        h2 = ((acc_ref[0] + acc_ref[1] + xwp_ref[...]) * dinv_ref[...]
              + b2_ref[...])
        t = jnp.dot(h2, wm1_ref[...],
                    preferred_element_type=jnp.float32) + bm1_ref[...]
        t = jnp.where(t > 0.0, t, jnp.exp(jnp.minimum(t, 0.0)) - 1.0)
        lg = jnp.dot(t, wm2_ref[...],
                     preferred_element_type=jnp.float32) + bm2_ref[...]
        m = jnp.max(lg, axis=-1, keepdims=True)
        ev = jnp.exp(lg - m)
        out_ref[...] = ev / jnp.sum(ev, axis=-1, keepdims=True)

    return pl.pallas_call(
        body,
        grid=grid,
        in_specs=[
            pl.BlockSpec((NC, RB, hh), lambda i: (0, i, 0)),
            pl.BlockSpec((RB, hh), lambda i: (i, 0)),
            pl.BlockSpec((RB, 1), lambda i: (i, 0)),
            pl.BlockSpec((1, hh), lambda i: (0, 0)),
            pl.BlockSpec((hh, hh), lambda i: (0, 0)),
            pl.BlockSpec((1, hh), lambda i: (0, 0)),
            pl.BlockSpec((hh, c), lambda i: (0, 0)),
            pl.BlockSpec((1, c), lambda i: (0, 0)),
        ],
        out_specs=pl.BlockSpec((RB, c), lambda i: (i, 0)),
        out_shape=jax.ShapeDtypeStruct((n, c), jnp.float32),
    )(acc, xwp, dinv, b2, wm1, bm1, wm2, bm2)


def kernel(X, edge_index, edge_weight, W1, b1, W2, b2, Wm1, bm1, Wm2, bm2):
    n, d = X.shape
    hh = W1.shape[1]
    c = Wm2.shape[1]
    e = edge_weight.shape[0]
    ept = e // NW

    c3 = edge_index[1].reshape(NW, ept // K, K)

    degp = _degree_partials(edge_index, edge_weight, n)
    xw1p, dinv = _tc_stage1(degp.T, X, W1)
    acc1 = _message_partials(xw1p, edge_index, c3, edge_weight, n, hh)
    xw2p = _tc_stage2(acc1, xw1p, dinv, b1.reshape(1, hh), W2)
    acc2 = _message_partials(xw2p, edge_index, c3, edge_weight, n, hh)
    return _tc_stage3(acc2, xw2p, dinv, b2.reshape(1, hh),
                      Wm1, bm1.reshape(1, hh), Wm2, bm2.reshape(1, c))
